# baseline (speedup 1.0000x reference)
import functools

import jax
import jax.numpy as jnp
from jax import lax
from jax.experimental import pallas as pl
from jax.experimental.pallas import tpu as pltpu
from jax.experimental.pallas import tpu_sc as plsc

_NUM_CORES = 2
_NUM_SUBCORES = 16
_NUM_WORKERS = _NUM_CORES * _NUM_SUBCORES
_LANES = 16


def kernel(user_id, interacted_items, user_table, item_table):
    del interacted_items, item_table
    batch = user_id.shape[0]
    dim = user_table.shape[1]
    b_per_w = batch // _NUM_WORKERS
    half = b_per_w // 2

    mesh = plsc.VectorSubcoreMesh(core_axis_name="c", subcore_axis_name="s")

    @functools.partial(
        pl.kernel,
        mesh=mesh,
        out_type=jax.ShapeDtypeStruct((batch, dim), jnp.float32),
        scratch_types=[
            pltpu.VMEM((b_per_w,), jnp.int32),
            pltpu.VMEM((half, dim), jnp.float32),
            pltpu.VMEM_SHARED((_NUM_SUBCORES, half, dim), jnp.float32),
            pltpu.SemaphoreType.DMA,
            pltpu.SemaphoreType.DMA,
        ],
    )
    def gather_rows(idx_hbm, table_hbm, out_hbm, idx_v, rows_v, rows_sh, sem_a, sem_b):
        sid = lax.axis_index("s")
        wid = sid * _NUM_CORES + lax.axis_index("c")
        base = wid * b_per_w
        pltpu.sync_copy(idx_hbm.at[pl.ds(base, b_per_w)], idx_v)

        def chunk_body(ci, carry):
            vec = idx_v[pl.ds(ci * _LANES, _LANES)]
            for j in range(_LANES):
                r = vec[j]
                row = ci * _LANES + j
                pltpu.make_async_copy(
                    table_hbm.at[pl.ds(r, 1), :],
                    rows_v.at[pl.ds(row, 1), :],
                    sem_a,
                ).start()
            return carry

        def chunk_body_sh(ci, carry):
            vec = idx_v[pl.ds(half + ci * _LANES, _LANES)]
            for j in range(_LANES):
                r = vec[j]
                row = ci * _LANES + j
                pltpu.make_async_copy(
                    table_hbm.at[pl.ds(r, 1), :],
                    rows_sh.at[sid, pl.ds(row, 1), :],
                    sem_b,
                ).start()
            return carry

        def both(ci, carry):
            chunk_body(ci, carry)
            chunk_body_sh(ci, carry)
            return carry

        lax.fori_loop(0, half // _LANES, both, 0)
        pltpu.make_async_copy(
            table_hbm.at[pl.ds(0, half), :], rows_v, sem_a
        ).wait()
        pltpu.make_async_copy(
            table_hbm.at[pl.ds(0, half), :], rows_sh.at[sid], sem_b
        ).wait()
        pltpu.sync_copy(rows_v, out_hbm.at[pl.ds(base, half)])
        pltpu.sync_copy(rows_sh.at[sid], out_hbm.at[pl.ds(base + half, half)])

    return gather_rows(user_id, user_table)
